# baseline (speedup 1.0000x reference)
import functools

import jax
import jax.numpy as jnp
from jax import lax
from jax.experimental import pallas as pl
from jax.experimental.pallas import tpu as pltpu
from jax.experimental.pallas import tpu_sc as plsc

NN = 10000
EE = 320000
DD = 128
HH = 64

NC = 2
NS = 16
NW = NC * NS
N_PAD = 10240
SL = N_PAD // NS
CW = 128
NCHUNK = EE // CW
CH_LO = NCHUNK // NW
NHI = NCHUNK - NW * CH_LO
CH_BUF = CH_LO + 1

_SC_MESH = plsc.VectorSubcoreMesh(core_axis_name="c", subcore_axis_name="s")
_SC_PARAMS = pltpu.CompilerParams(use_tc_tiling_on_sc=False)


@functools.partial(
    pl.kernel,
    out_type=jax.ShapeDtypeStruct((NC, N_PAD, 16), jnp.float32),
    mesh=_SC_MESH,
    compiler_params=_SC_PARAMS,
    scratch_types=[
        pltpu.VMEM((CH_BUF, CW), jnp.int32),
        pltpu.VMEM((CW, 16), jnp.float32),
        pltpu.VMEM_SHARED((N_PAD, 16), jnp.float32),
        pltpu.SemaphoreType.DMA,
    ],
)
def _deg_kernel(e3_hbm, ones_hbm, zeros_hbm, out_hbm, dstv, onesv, acc, ssem):
    c = lax.axis_index("c")
    s = lax.axis_index("s")
    wid = c * NS + s
    base = CH_LO * wid + jnp.maximum(wid - (NW - NHI), 0)
    nch = CH_LO + jnp.where(wid >= NW - NHI, 1, 0)
    pltpu.sync_copy(e3_hbm.at[1, pl.ds(base, CH_BUF)], dstv)
    pltpu.sync_copy(ones_hbm, onesv)
    pltpu.sync_copy(zeros_hbm, acc.at[pl.ds(s * SL, SL)])
    plsc.subcore_barrier()

    LAG = 16

    def ss(j):
        pltpu.async_copy(onesv, acc.at[dstv.at[j]], ssem, add=True)

    def sw(j):
        pltpu.make_async_copy(onesv, acc.at[dstv.at[j]], ssem).wait()

    for j in range(LAG):
        ss(j)

    def body(k, carry):
        ss(k + LAG)
        sw(k)
        return carry

    lax.fori_loop(0, nch - LAG, body, 0)
    for t in range(LAG):
        sw(nch - LAG + t)
    plsc.subcore_barrier()
    pltpu.sync_copy(acc.at[pl.ds(s * SL, SL)], out_hbm.at[c, pl.ds(s * SL, SL)])


def _make_agg(write_dinvp):
    out_types = [
        jax.ShapeDtypeStruct((NC, N_PAD, HH), jnp.float32),
        jax.ShapeDtypeStruct((NC, N_PAD, HH), jnp.float32),
    ]
    scratch = [
        pltpu.VMEM((CH_BUF, CW), jnp.int32),
        pltpu.VMEM((CH_BUF, CW), jnp.int32),
        pltpu.VMEM((4, CW, HH), jnp.float32),
        pltpu.VMEM((2, 128, HH), jnp.float32),
        pltpu.VMEM((2, 128, 16), jnp.float32),
        pltpu.VMEM_SHARED((N_PAD, HH), jnp.float32),
        pltpu.SemaphoreType.DMA,
        pltpu.SemaphoreType.DMA,
    ]
    if write_dinvp:
        out_types = out_types + [jax.ShapeDtypeStruct((N_PAD, HH), jnp.float32)]
        scratch = scratch + [pltpu.VMEM((128, HH), jnp.float32)]
    return pl.kernel(
        functools.partial(_agg_body, write_dinvp),
        out_type=out_types,
        mesh=_SC_MESH,
        compiler_params=_SC_PARAMS,
        scratch_types=scratch,
    )


def _agg_body(write_dinvp, u_hbm, dinv_hbm, e3_hbm, zeros_hbm, out_hbm,
              ytmp_hbm, *rest):
    if write_dinvp:
        dinvp_hbm, srcv, dstv, rows, ybuf, dbuf, acc, gsem, ssem, dpbuf = rest
    else:
        srcv, dstv, rows, ybuf, dbuf, acc, gsem, ssem = rest
    c = lax.axis_index("c")
    s = lax.axis_index("s")
    wid = c * NS + s
    base = CH_LO * wid + jnp.maximum(wid - (NW - NHI), 0)
    nch = CH_LO + jnp.where(wid >= NW - NHI, 1, 0)
    pltpu.sync_copy(e3_hbm.at[0, pl.ds(base, CH_BUF)], srcv)
    pltpu.sync_copy(e3_hbm.at[1, pl.ds(base, CH_BUF)], dstv)

    NT = SL // 128

    def p1_load(t):
        r0 = s * SL + t * 128
        b = t % 2
        pltpu.async_copy(u_hbm.at[pl.ds(r0, 128)], ybuf.at[b], gsem)
        pltpu.async_copy(dinv_hbm.at[pl.ds(r0, 128)], dbuf.at[b], gsem)

    def p1_wait(t):
        r0 = s * SL + t * 128
        b = t % 2
        pltpu.make_async_copy(u_hbm.at[pl.ds(r0, 128)], ybuf.at[b], gsem).wait()
        pltpu.make_async_copy(dinv_hbm.at[pl.ds(r0, 128)], dbuf.at[b], gsem).wait()

    p1_load(0)
    for t in range(NT):
        b = t % 2
        r0 = s * SL + t * 128
        if t + 1 < NT:
            p1_load(t + 1)
        p1_wait(t)

        def scale_row(i, carry, b=b):
            dv = dbuf[b, i, :]
            for q in range(HH // 16):
                ybuf[b, i, pl.ds(q * 16, 16)] = ybuf[b, i, pl.ds(q * 16, 16)] * dv
                if write_dinvp:
                    dpbuf[i, pl.ds(q * 16, 16)] = dv
            return carry

        lax.fori_loop(0, 128, scale_row, 0)
        pltpu.sync_copy(ybuf.at[b], ytmp_hbm.at[c, pl.ds(r0, 128)])

        @pl.when(c == 0)
        def _(b=b, r0=r0):
            pltpu.sync_copy(ybuf.at[b], acc.at[pl.ds(r0, 128)])
            if write_dinvp:
                pltpu.sync_copy(dpbuf, dinvp_hbm.at[pl.ds(r0, 128)])

    @pl.when(c != 0)
    def _():
        pltpu.sync_copy(zeros_hbm, acc.at[pl.ds(s * SL, SL)])

    plsc.subcore_barrier()

    def gs(j):
        pltpu.async_copy(ytmp_hbm.at[c].at[srcv.at[j]], rows.at[lax.rem(j, 4)],
                         gsem)

    def gw(j):
        pltpu.make_async_copy(
            ytmp_hbm.at[c].at[srcv.at[j]], rows.at[lax.rem(j, 4)], gsem).wait()

    def ss(j):
        pltpu.async_copy(rows.at[lax.rem(j, 4)], acc.at[dstv.at[j]], ssem,
                         add=True)

    def sw(j):
        pltpu.make_async_copy(
            rows.at[lax.rem(j, 4)], acc.at[dstv.at[j]], ssem).wait()

    gs(0)
    gs(1)
    gw(0)
    ss(0)
    gs(2)
    gw(1)
    ss(1)
    gs(3)

    def body(k, carry):
        j = k + 2
        gw(j)
        ss(j)
        sw(j - 2)
        gs(j + 2)
        return carry

    lax.fori_loop(0, nch - 4, body, 0)
    for t in range(2):
        j = nch - 2 + t
        gw(j)
        ss(j)
        sw(j - 2)
    sw(nch - 2)
    sw(nch - 1)
    plsc.subcore_barrier()

    pltpu.sync_copy(acc.at[pl.ds(s * SL, SL)], out_hbm.at[c, pl.ds(s * SL, SL)])


_agg1 = _make_agg(True)
_agg2 = _make_agg(False)


R = 10240
GRID = N_PAD // R
RP = R // 2


def _blockdiag2(w):
    a, b = w.shape
    z = jnp.zeros((a, b), w.dtype)
    return jnp.concatenate(
        [jnp.concatenate([w, z], axis=1), jnp.concatenate([z, w], axis=1)],
        axis=0)


def _t1_body(x_ref, w1_ref, dp_ref, u_ref, dinv_ref):
    dp = dp_ref[...]
    deg = dp[0] + dp[1] + 1.0
    dinv_ref[...] = lax.rsqrt(deg)
    u_ref[...] = jnp.dot(x_ref[...], w1_ref[...],
                         preferred_element_type=jnp.float32)


_t1 = pl.pallas_call(
    _t1_body,
    grid=(GRID,),
    in_specs=[
        pl.BlockSpec((RP, 2 * DD), lambda i: (i, 0)),
        pl.BlockSpec((2 * DD, 2 * HH), lambda i: (0, 0)),
        pl.BlockSpec((NC, R // 8, 128), lambda i: (0, i, 0)),
    ],
    out_specs=[
        pl.BlockSpec((RP, 2 * HH), lambda i: (i, 0)),
        pl.BlockSpec((R // 8, 128), lambda i: (i, 0)),
    ],
    out_shape=[
        jax.ShapeDtypeStruct((N_PAD // 2, 2 * HH), jnp.float32),
        jax.ShapeDtypeStruct((N_PAD // 8, 128), jnp.float32),
    ],
)


def _t2_body(q_ref, dp_ref, b1_ref, w2_ref, u2_ref):
    q = q_ref[...]
    h = jnp.maximum((q[0] + q[1]) * dp_ref[...] + b1_ref[...], 0.0)
    u2_ref[...] = jnp.dot(h, w2_ref[...], preferred_element_type=jnp.float32)


_t2 = pl.pallas_call(
    _t2_body,
    grid=(GRID,),
    in_specs=[
        pl.BlockSpec((NC, RP, 2 * HH), lambda i: (0, i, 0)),
        pl.BlockSpec((RP, 2 * HH), lambda i: (i, 0)),
        pl.BlockSpec((1, 2 * HH), lambda i: (0, 0)),
        pl.BlockSpec((2 * HH, 2 * HH), lambda i: (0, 0)),
    ],
    out_specs=pl.BlockSpec((RP, 2 * HH), lambda i: (i, 0)),
    out_shape=jax.ShapeDtypeStruct((N_PAD // 2, 2 * HH), jnp.float32),
)


def _t3_body(q_ref, dp_ref, b2_ref, fw1_ref, fb1_ref, fw2_ref, fb2_ref,
             fw3_ref, fb3_ref, out_ref):
    q = q_ref[...]
    h = jnp.maximum((q[0] + q[1]) * dp_ref[...] + b2_ref[...], 0.0)
    h = jnp.maximum(
        jnp.dot(h, fw1_ref[...], preferred_element_type=jnp.float32) + fb1_ref[...], 0.0)
    h = jnp.maximum(
        jnp.dot(h, fw2_ref[...], preferred_element_type=jnp.float32) + fb2_ref[...], 0.0)
    out_t = lax.dot_general(fw3_ref[...], h, (((0,), (1,)), ((), ())),
                            preferred_element_type=jnp.float32)
    out_ref[...] = out_t + fb3_ref[...]


_t3 = pl.pallas_call(
    _t3_body,
    grid=(GRID,),
    in_specs=[
        pl.BlockSpec((NC, RP, 2 * HH), lambda i: (0, i, 0)),
        pl.BlockSpec((RP, 2 * HH), lambda i: (i, 0)),
        pl.BlockSpec((1, 2 * HH), lambda i: (0, 0)),
        pl.BlockSpec((2 * HH, 2 * HH), lambda i: (0, 0)),
        pl.BlockSpec((1, 2 * HH), lambda i: (0, 0)),
        pl.BlockSpec((2 * HH, 2 * HH), lambda i: (0, 0)),
        pl.BlockSpec((1, 2 * HH), lambda i: (0, 0)),
        pl.BlockSpec((2 * HH, 8), lambda i: (0, 0)),
        pl.BlockSpec((8, 1), lambda i: (0, 0)),
    ],
    out_specs=pl.BlockSpec((8, RP), lambda i: (0, i)),
    out_shape=jax.ShapeDtypeStruct((8, N_PAD // 2), jnp.float32),
)


def kernel(x, edge_index, W1, b1, W2, b2, fW1, fb1, fW2, fb2, fW3, fb3):
    xp = jnp.pad(x, ((0, N_PAD - NN), (0, 0))).reshape(N_PAD // 2, 2 * DD)
    e3 = edge_index.reshape(2, NCHUNK, CW)
    ones16 = jnp.ones((CW, 16), jnp.float32)
    z16 = jnp.zeros((SL, 16), jnp.float32)
    z64 = jnp.zeros((SL, HH), jnp.float32)

    W1d = _blockdiag2(W1)
    W2d = _blockdiag2(W2)
    fW1d = _blockdiag2(fW1)
    fW2d = _blockdiag2(fW2)
    b1_2 = jnp.concatenate([b1, b1]).reshape(1, 2 * HH)
    b2_2 = jnp.concatenate([b2, b2]).reshape(1, 2 * HH)
    fb1_2 = jnp.concatenate([fb1, fb1]).reshape(1, 2 * HH)
    fb2_2 = jnp.concatenate([fb2, fb2]).reshape(1, 2 * HH)
    fW3d = jnp.zeros((2 * HH, 8), jnp.float32)
    fW3d = fW3d.at[:HH, 0:1].set(fW3).at[HH:, 1:2].set(fW3)
    fb3d = jnp.zeros((8, 1), jnp.float32).at[0:2, 0].set(fb3[0])

    degp = _deg_kernel(e3, ones16, z16)
    u1p, dinv8 = _t1(xp, W1d, degp.reshape(NC, N_PAD // 8, 128))
    dinv16 = dinv8.reshape(N_PAD, 16)
    q1, _yt1, dinvp = _agg1(u1p.reshape(N_PAD, HH), dinv16, e3, z64)
    dpk = dinvp.reshape(N_PAD // 2, 2 * HH)
    u2p = _t2(q1.reshape(NC, N_PAD // 2, 2 * HH), dpk, b1_2, W2d)
    q2, _yt2 = _agg2(u2p.reshape(N_PAD, HH), dinv16, e3, z64)
    outt = _t3(q2.reshape(NC, N_PAD // 2, 2 * HH), dpk, b2_2, fW1d, fb1_2,
               fW2d, fb2_2, fW3d, fb3d)
    out = jnp.stack([outt[0], outt[1]], axis=-1).reshape(N_PAD)
    return out[:NN]


# speedup vs baseline: 1.0119x; 1.0119x over previous
import functools

import jax
import jax.numpy as jnp
from jax import lax
from jax.experimental import pallas as pl
from jax.experimental.pallas import tpu as pltpu
from jax.experimental.pallas import tpu_sc as plsc

NN = 10000
EE = 320000
DD = 128
HH = 64

NC = 2
NS = 16
NW = NC * NS
N_PAD = 10240
SL = N_PAD // NS
CW = 128
NCHUNK = EE // CW
CH_LO = NCHUNK // NW
NHI = NCHUNK - NW * CH_LO
CH_BUF = CH_LO + 1

_SC_MESH = plsc.VectorSubcoreMesh(core_axis_name="c", subcore_axis_name="s")
_SC_PARAMS = pltpu.CompilerParams(use_tc_tiling_on_sc=False)


@functools.partial(
    pl.kernel,
    out_type=jax.ShapeDtypeStruct((NC, N_PAD, 16), jnp.float32),
    mesh=_SC_MESH,
    compiler_params=_SC_PARAMS,
    scratch_types=[
        pltpu.VMEM((CH_BUF, CW), jnp.int32),
        pltpu.VMEM((CW, 16), jnp.float32),
        pltpu.VMEM_SHARED((N_PAD, 16), jnp.float32),
        pltpu.SemaphoreType.DMA,
    ],
)
def _deg_kernel(e3_hbm, ones_hbm, zeros_hbm, out_hbm, dstv, onesv, acc, ssem):
    c = lax.axis_index("c")
    s = lax.axis_index("s")
    wid = c * NS + s
    base = CH_LO * wid + jnp.maximum(wid - (NW - NHI), 0)
    nch = CH_LO + jnp.where(wid >= NW - NHI, 1, 0)
    pltpu.sync_copy(e3_hbm.at[1, pl.ds(base, CH_BUF)], dstv)
    pltpu.sync_copy(ones_hbm, onesv)
    pltpu.sync_copy(zeros_hbm, acc.at[pl.ds(s * SL, SL)])
    plsc.subcore_barrier()

    LAG = 16

    def ss(j):
        pltpu.async_copy(onesv, acc.at[dstv.at[j]], ssem, add=True)

    def sw(j):
        pltpu.make_async_copy(onesv, acc.at[dstv.at[j]], ssem).wait()

    for j in range(LAG):
        ss(j)

    def body(k, carry):
        ss(k + LAG)
        sw(k)
        return carry

    lax.fori_loop(0, nch - LAG, body, 0)
    for t in range(LAG):
        sw(nch - LAG + t)
    plsc.subcore_barrier()
    pltpu.sync_copy(acc.at[pl.ds(s * SL, SL)], out_hbm.at[c, pl.ds(s * SL, SL)])


def _make_agg(write_dinvp):
    out_types = [
        jax.ShapeDtypeStruct((NC, N_PAD, HH), jnp.float32),
        jax.ShapeDtypeStruct((NC, N_PAD, HH), jnp.float32),
    ]
    scratch = [
        pltpu.VMEM((CH_BUF, CW), jnp.int32),
        pltpu.VMEM((CH_BUF, CW), jnp.int32),
        pltpu.VMEM((4, CW, HH), jnp.float32),
        pltpu.VMEM((2, 128, HH), jnp.float32),
        pltpu.VMEM((2, 128, 16), jnp.float32),
        pltpu.VMEM_SHARED((N_PAD, HH), jnp.float32),
        pltpu.SemaphoreType.DMA,
        pltpu.SemaphoreType.DMA,
    ]
    if write_dinvp:
        out_types = out_types + [jax.ShapeDtypeStruct((N_PAD, HH), jnp.float32)]
        scratch = scratch + [pltpu.VMEM((128, HH), jnp.float32)]
    return pl.kernel(
        functools.partial(_agg_body, write_dinvp),
        out_type=out_types,
        mesh=_SC_MESH,
        compiler_params=_SC_PARAMS,
        scratch_types=scratch,
    )


def _agg_body(write_dinvp, u_hbm, dinv_hbm, e3_hbm, zeros_hbm, out_hbm,
              ytmp_hbm, *rest):
    if write_dinvp:
        dinvp_hbm, srcv, dstv, rows, ybuf, dbuf, acc, gsem, ssem, dpbuf = rest
    else:
        srcv, dstv, rows, ybuf, dbuf, acc, gsem, ssem = rest
    c = lax.axis_index("c")
    s = lax.axis_index("s")
    wid = c * NS + s
    base = CH_LO * wid + jnp.maximum(wid - (NW - NHI), 0)
    nch = CH_LO + jnp.where(wid >= NW - NHI, 1, 0)
    pltpu.sync_copy(e3_hbm.at[0, pl.ds(base, CH_BUF)], srcv)
    pltpu.sync_copy(e3_hbm.at[1, pl.ds(base, CH_BUF)], dstv)

    NT = SL // 128

    def p1_load(t):
        r0 = s * SL + t * 128
        b = t % 2
        pltpu.async_copy(u_hbm.at[pl.ds(r0, 128)], ybuf.at[b], gsem)
        pltpu.async_copy(dinv_hbm.at[pl.ds(r0, 128)], dbuf.at[b], gsem)

    def p1_wait(t):
        r0 = s * SL + t * 128
        b = t % 2
        pltpu.make_async_copy(u_hbm.at[pl.ds(r0, 128)], ybuf.at[b], gsem).wait()
        pltpu.make_async_copy(dinv_hbm.at[pl.ds(r0, 128)], dbuf.at[b], gsem).wait()

    p1_load(0)
    for t in range(NT):
        b = t % 2
        r0 = s * SL + t * 128
        if t + 1 < NT:
            p1_load(t + 1)
        p1_wait(t)

        def scale_row(i, carry, b=b):
            dv = dbuf[b, i, :]
            for q in range(HH // 16):
                ybuf[b, i, pl.ds(q * 16, 16)] = ybuf[b, i, pl.ds(q * 16, 16)] * dv
                if write_dinvp:
                    dpbuf[i, pl.ds(q * 16, 16)] = dv
            return carry

        lax.fori_loop(0, 128, scale_row, 0)
        pltpu.sync_copy(ybuf.at[b], ytmp_hbm.at[c, pl.ds(r0, 128)])

        @pl.when(c == 0)
        def _(b=b, r0=r0):
            pltpu.sync_copy(ybuf.at[b], acc.at[pl.ds(r0, 128)])
            if write_dinvp:
                pltpu.sync_copy(dpbuf, dinvp_hbm.at[pl.ds(r0, 128)])

    @pl.when(c != 0)
    def _():
        pltpu.sync_copy(zeros_hbm, acc.at[pl.ds(s * SL, SL)])

    plsc.subcore_barrier()

    def gs(j):
        pltpu.async_copy(ytmp_hbm.at[c].at[srcv.at[j]], rows.at[lax.rem(j, 4)],
                         gsem)

    def gw(j):
        pltpu.make_async_copy(
            ytmp_hbm.at[c].at[srcv.at[j]], rows.at[lax.rem(j, 4)], gsem).wait()

    def ss(j):
        pltpu.async_copy(rows.at[lax.rem(j, 4)], acc.at[dstv.at[j]], ssem,
                         add=True)

    def sw(j):
        pltpu.make_async_copy(
            rows.at[lax.rem(j, 4)], acc.at[dstv.at[j]], ssem).wait()

    gs(0)
    gs(1)
    gw(0)
    ss(0)
    gs(2)
    gw(1)
    ss(1)
    gs(3)

    def body(k, carry):
        j = k + 2
        gw(j)
        ss(j)
        sw(j - 2)
        gs(j + 2)
        return carry

    lax.fori_loop(0, nch - 4, body, 0)
    for t in range(2):
        j = nch - 2 + t
        gw(j)
        ss(j)
        sw(j - 2)
    sw(nch - 2)
    sw(nch - 1)
    plsc.subcore_barrier()

    pltpu.sync_copy(acc.at[pl.ds(s * SL, SL)], out_hbm.at[c, pl.ds(s * SL, SL)])


_agg1 = _make_agg(True)
_agg2 = _make_agg(False)


R = 5120
GRID = N_PAD // R
RP = R // 2


def _blockdiag2(w):
    a, b = w.shape
    z = jnp.zeros((a, b), w.dtype)
    return jnp.concatenate(
        [jnp.concatenate([w, z], axis=1), jnp.concatenate([z, w], axis=1)],
        axis=0)


def _t1_body(x_ref, w1_ref, dp_ref, u_ref, dinv_ref):
    dp = dp_ref[...]
    deg = dp[0] + dp[1] + 1.0
    dinv_ref[...] = lax.rsqrt(deg)
    u_ref[...] = jnp.dot(x_ref[...], w1_ref[...],
                         preferred_element_type=jnp.float32)


_t1 = pl.pallas_call(
    _t1_body,
    grid=(GRID,),
    in_specs=[
        pl.BlockSpec((RP, 2 * DD), lambda i: (i, 0)),
        pl.BlockSpec((2 * DD, 2 * HH), lambda i: (0, 0)),
        pl.BlockSpec((NC, R // 8, 128), lambda i: (0, i, 0)),
    ],
    out_specs=[
        pl.BlockSpec((RP, 2 * HH), lambda i: (i, 0)),
        pl.BlockSpec((R // 8, 128), lambda i: (i, 0)),
    ],
    out_shape=[
        jax.ShapeDtypeStruct((N_PAD // 2, 2 * HH), jnp.float32),
        jax.ShapeDtypeStruct((N_PAD // 8, 128), jnp.float32),
    ],
)


def _t2_body(q_ref, dp_ref, b1_ref, w2_ref, u2_ref):
    q = q_ref[...]
    h = jnp.maximum((q[0] + q[1]) * dp_ref[...] + b1_ref[...], 0.0)
    u2_ref[...] = jnp.dot(h, w2_ref[...], preferred_element_type=jnp.float32)


_t2 = pl.pallas_call(
    _t2_body,
    grid=(GRID,),
    in_specs=[
        pl.BlockSpec((NC, RP, 2 * HH), lambda i: (0, i, 0)),
        pl.BlockSpec((RP, 2 * HH), lambda i: (i, 0)),
        pl.BlockSpec((1, 2 * HH), lambda i: (0, 0)),
        pl.BlockSpec((2 * HH, 2 * HH), lambda i: (0, 0)),
    ],
    out_specs=pl.BlockSpec((RP, 2 * HH), lambda i: (i, 0)),
    out_shape=jax.ShapeDtypeStruct((N_PAD // 2, 2 * HH), jnp.float32),
)


def _t3_body(q_ref, dp_ref, b2_ref, fw1_ref, fb1_ref, fw2_ref, fb2_ref,
             fw3_ref, fb3_ref, out_ref):
    q = q_ref[...]
    h = jnp.maximum((q[0] + q[1]) * dp_ref[...] + b2_ref[...], 0.0)
    h = jnp.maximum(
        jnp.dot(h, fw1_ref[...], preferred_element_type=jnp.float32) + fb1_ref[...], 0.0)
    h = jnp.maximum(
        jnp.dot(h, fw2_ref[...], preferred_element_type=jnp.float32) + fb2_ref[...], 0.0)
    out_t = lax.dot_general(fw3_ref[...], h, (((0,), (1,)), ((), ())),
                            preferred_element_type=jnp.float32)
    out_ref[...] = out_t + fb3_ref[...]


_t3 = pl.pallas_call(
    _t3_body,
    grid=(GRID,),
    in_specs=[
        pl.BlockSpec((NC, RP, 2 * HH), lambda i: (0, i, 0)),
        pl.BlockSpec((RP, 2 * HH), lambda i: (i, 0)),
        pl.BlockSpec((1, 2 * HH), lambda i: (0, 0)),
        pl.BlockSpec((2 * HH, 2 * HH), lambda i: (0, 0)),
        pl.BlockSpec((1, 2 * HH), lambda i: (0, 0)),
        pl.BlockSpec((2 * HH, 2 * HH), lambda i: (0, 0)),
        pl.BlockSpec((1, 2 * HH), lambda i: (0, 0)),
        pl.BlockSpec((2 * HH, 8), lambda i: (0, 0)),
        pl.BlockSpec((8, 1), lambda i: (0, 0)),
    ],
    out_specs=pl.BlockSpec((8, RP), lambda i: (0, i)),
    out_shape=jax.ShapeDtypeStruct((8, N_PAD // 2), jnp.float32),
)


def kernel(x, edge_index, W1, b1, W2, b2, fW1, fb1, fW2, fb2, fW3, fb3):
    xp = jnp.pad(x, ((0, N_PAD - NN), (0, 0))).reshape(N_PAD // 2, 2 * DD)
    e3 = edge_index.reshape(2, NCHUNK, CW)
    ones16 = jnp.ones((CW, 16), jnp.float32)
    z16 = jnp.zeros((SL, 16), jnp.float32)
    z64 = jnp.zeros((SL, HH), jnp.float32)

    W1d = _blockdiag2(W1)
    W2d = _blockdiag2(W2)
    fW1d = _blockdiag2(fW1)
    fW2d = _blockdiag2(fW2)
    b1_2 = jnp.concatenate([b1, b1]).reshape(1, 2 * HH)
    b2_2 = jnp.concatenate([b2, b2]).reshape(1, 2 * HH)
    fb1_2 = jnp.concatenate([fb1, fb1]).reshape(1, 2 * HH)
    fb2_2 = jnp.concatenate([fb2, fb2]).reshape(1, 2 * HH)
    fW3d = jnp.zeros((2 * HH, 8), jnp.float32)
    fW3d = fW3d.at[:HH, 0:1].set(fW3).at[HH:, 1:2].set(fW3)
    fb3d = jnp.zeros((8, 1), jnp.float32).at[0:2, 0].set(fb3[0])

    degp = _deg_kernel(e3, ones16, z16)
    u1p, dinv8 = _t1(xp, W1d, degp.reshape(NC, N_PAD // 8, 128))
    dinv16 = dinv8.reshape(N_PAD, 16)
    q1, _yt1, dinvp = _agg1(u1p.reshape(N_PAD, HH), dinv16, e3, z64)
    dpk = dinvp.reshape(N_PAD // 2, 2 * HH)
    u2p = _t2(q1.reshape(NC, N_PAD // 2, 2 * HH), dpk, b1_2, W2d)
    q2, _yt2 = _agg2(u2p.reshape(N_PAD, HH), dinv16, e3, z64)
    outt = _t3(q2.reshape(NC, N_PAD // 2, 2 * HH), dpk, b2_2, fW1d, fb1_2,
               fW2d, fb2_2, fW3d, fb3d)
    out = jnp.stack([outt[0], outt[1]], axis=-1).reshape(N_PAD)
    return out[:NN]
